# baseline (speedup 1.0000x reference)
import functools

import jax
import jax.numpy as jnp
from jax.experimental import pallas as pl

BF16 = jnp.bfloat16
F32 = jnp.float32


def _dot(a, b):
    return jax.lax.dot_general(a, b, (((1,), (0,)), ((), ())),
                               preferred_element_type=F32)


def _accum_rows(win, w, o_ref, bias, relu, TR, Cout, W, out_dtype):
    Cin = win.shape[1]
    zc = jnp.zeros((Cout, 1), F32)
    for r in range(TR):
        rhs = win[r:r + 3].reshape(3 * Cin, W)
        f0 = _dot(w[0], rhs)
        f1 = _dot(w[1], rhs)
        f2 = _dot(w[2], rhs)
        acc = (jnp.concatenate([zc, f0[:, :W - 1]], axis=1) + f1
               + jnp.concatenate([f2[:, 1:], zc], axis=1) + bias)
        if relu:
            acc = jnp.maximum(acc, 0.0)
        o_ref[0, r] = acc.astype(out_dtype)


def _conv_s1_body(*refs, nin, TR, W, Cout, relu, out_dtype):
    o_ref = refs[-1]
    b_ref = refs[-2]
    wins = []
    for k in range(nin):
        top, cur, bot = refs[3 * k], refs[3 * k + 1], refs[3 * k + 2]
        wins.append(jnp.concatenate([top[0], cur[0], bot[0]], axis=0))
    win = wins[0] if nin == 1 else jnp.concatenate(wins, axis=1)
    _accum_rows(win, refs[3 * nin], o_ref, b_ref[...], relu, TR, Cout, W,
                out_dtype)


def _halos(x, TR):
    B, H, C, W = x.shape
    z = jnp.zeros((B, 1, C, W), x.dtype)
    top = jnp.concatenate([z, x[:, TR - 1::TR][:, :-1]], axis=1)
    bot = jnp.concatenate([x[:, TR::TR], z], axis=1)
    return top, bot


def _conv_s1(xs, w, b, relu, TR, out_dtype=BF16):
    B, H, _, W = xs[0].shape
    Cout = w.shape[1]
    nb = H // TR
    args, in_specs = [], []
    for x in xs:
        Cin = x.shape[2]
        top, bot = _halos(x, TR)
        args += [top, x, bot]
        in_specs += [
            pl.BlockSpec((1, 1, Cin, W), lambda bb, i: (bb, i, 0, 0)),
            pl.BlockSpec((1, TR, Cin, W), lambda bb, i: (bb, i, 0, 0)),
            pl.BlockSpec((1, 1, Cin, W), lambda bb, i: (bb, i, 0, 0)),
        ]
    args += [w, b]
    in_specs += [pl.BlockSpec(w.shape, lambda bb, i: (0, 0, 0)),
                 pl.BlockSpec(b.shape, lambda bb, i: (0, 0))]
    return pl.pallas_call(
        functools.partial(_conv_s1_body, nin=len(xs), TR=TR, W=W, Cout=Cout,
                          relu=relu, out_dtype=out_dtype),
        grid=(B, nb),
        in_specs=in_specs,
        out_specs=pl.BlockSpec((1, TR, Cout, W), lambda bb, i: (bb, i, 0, 0)),
        out_shape=jax.ShapeDtypeStruct((B, H, Cout, W), out_dtype),
    )(*args)


def _conv_s2_body(xc, xb, s0, s1, w, b_ref, o_ref, *, TR, Wo, Cin, Cout):
    W = 2 * Wo
    z = jnp.zeros((1, Cin, W), BF16)
    win = jnp.concatenate([xc[0], xb[0], z], axis=0)
    flat = win.reshape((2 * TR + 2) * Cin, W)
    wine = _dot(flat, s0[...]).astype(BF16).reshape(2 * TR + 2, Cin, Wo)
    wino = _dot(flat, s1[...]).astype(BF16).reshape(2 * TR + 2, Cin, Wo)
    zc = jnp.zeros((Cout, 1), F32)
    for r in range(TR):
        rhs_e = wine[2 * r:2 * r + 3].reshape(3 * Cin, Wo)
        rhs_o = wino[2 * r:2 * r + 3].reshape(3 * Cin, Wo)
        f0 = _dot(w[0], rhs_e)
        f1 = _dot(w[1], rhs_o)
        f2 = _dot(w[2], rhs_e)
        acc = f0 + f1 + jnp.concatenate([f2[:, 1:], zc], axis=1) + b_ref[...]
        acc = jnp.maximum(acc, 0.0)
        o_ref[0, r] = acc.astype(BF16)


def _conv_s2(x, w, b, TR):
    B, H, Cin, W = x.shape
    Cout = w.shape[1]
    Ho, Wo = H // 2, W // 2
    nb = Ho // TR
    zb = jnp.zeros((B, 1, Cin, W), BF16)
    bot = jnp.concatenate([x[:, 2 * TR::2 * TR], zb], axis=1)
    cols = jnp.arange(W)[:, None]
    s0 = (cols == 2 * jnp.arange(Wo)[None, :]).astype(BF16)
    s1 = (cols == 2 * jnp.arange(Wo)[None, :] + 1).astype(BF16)
    return pl.pallas_call(
        functools.partial(_conv_s2_body, TR=TR, Wo=Wo, Cin=Cin, Cout=Cout),
        grid=(B, nb),
        in_specs=[pl.BlockSpec((1, 2 * TR, Cin, W), lambda bb, i: (bb, i, 0, 0)),
                  pl.BlockSpec((1, 1, Cin, W), lambda bb, i: (bb, i, 0, 0)),
                  pl.BlockSpec(s0.shape, lambda bb, i: (0, 0)),
                  pl.BlockSpec(s1.shape, lambda bb, i: (0, 0)),
                  pl.BlockSpec(w.shape, lambda bb, i: (0, 0, 0)),
                  pl.BlockSpec(b.shape, lambda bb, i: (0, 0))],
        out_specs=pl.BlockSpec((1, TR, Cout, Wo), lambda bb, i: (bb, i, 0, 0)),
        out_shape=jax.ShapeDtypeStruct((B, Ho, Cout, Wo), BF16),
    )(x, bot, s0, s1, w, b)


def _conv_up_body(top, cur, bot, rmat, w, b_ref, o_ref, *, TR2, Wd, Cin, Cout):
    TR, W = 2 * TR2, 2 * Wd
    dwin = jnp.concatenate([top[0], cur[0], bot[0]], axis=0)
    flat = dwin.reshape((TR2 + 2) * Cin, Wd)
    wide = _dot(flat, rmat[...]).astype(BF16).reshape(TR2 + 2, Cin, W)
    urep = jnp.repeat(wide, 2, axis=0)[1:TR + 3]
    _accum_rows(urep, w, o_ref, b_ref[...], True, TR, Cout, W, BF16)


def _conv_up(d, w, b, TR2):
    B, Hd, Cin, Wd = d.shape
    Cout = w.shape[1]
    H, W = 2 * Hd, 2 * Wd
    TR = 2 * TR2
    nb = Hd // TR2
    top, bot = _halos(d, TR2)
    rmat = (jnp.arange(Wd)[:, None] == (jnp.arange(W)[None, :] // 2)).astype(BF16)
    return pl.pallas_call(
        functools.partial(_conv_up_body, TR2=TR2, Wd=Wd, Cin=Cin, Cout=Cout),
        grid=(B, nb),
        in_specs=[pl.BlockSpec((1, 1, Cin, Wd), lambda bb, i: (bb, i, 0, 0)),
                  pl.BlockSpec((1, TR2, Cin, Wd), lambda bb, i: (bb, i, 0, 0)),
                  pl.BlockSpec((1, 1, Cin, Wd), lambda bb, i: (bb, i, 0, 0)),
                  pl.BlockSpec(rmat.shape, lambda bb, i: (0, 0)),
                  pl.BlockSpec(w.shape, lambda bb, i: (0, 0, 0)),
                  pl.BlockSpec(b.shape, lambda bb, i: (0, 0))],
        out_specs=pl.BlockSpec((1, TR, Cout, W), lambda bb, i: (bb, i, 0, 0)),
        out_shape=jax.ShapeDtypeStruct((B, H, Cout, W), BF16),
    )(top, d, bot, rmat, w, b)


def _wk(w):
    O, I = w.shape[0], w.shape[1]
    return jnp.transpose(w, (3, 0, 2, 1)).reshape(3, O, 3 * I).astype(BF16)


def kernel(image, w1, b1, w2, b2, w3, b3, w4, b4, w5, b5, w6, b6, w_out, b_out):
    x = jnp.transpose(image, (0, 2, 1, 3)).astype(BF16)
    bb = lambda b: b.reshape(-1, 1).astype(F32)
    t = _conv_s1([x], _wk(w1), bb(b1), True, 32)
    e1 = _conv_s1([t], _wk(w2), bb(b2), True, 32)
    d = _conv_s2(e1, _wk(w3), bb(b3), 32)
    d = _conv_s1([d], _wk(w4), bb(b4), True, 32)
    u = _conv_up(d, _wk(w5), bb(b5), 16)
    f = _conv_s1([u, e1], _wk(w6), bb(b6), True, 32)
    y = _conv_s1([f], _wk(w_out), bb(b_out), True, 32, out_dtype=F32)
    return jnp.transpose(y, (0, 2, 1, 3))


# speedup vs baseline: 1.0259x; 1.0259x over previous
import functools

import jax
import jax.numpy as jnp
from jax.experimental import pallas as pl

BF16 = jnp.bfloat16
F32 = jnp.float32


def _dot(a, b):
    return jax.lax.dot_general(a, b, (((1,), (0,)), ((), ())),
                               preferred_element_type=F32)


def _shift3(win, W):
    z = jnp.zeros(win.shape[:2] + (1,), win.dtype)
    return [jnp.concatenate([z, win[:, :, :W - 1]], axis=2),
            win,
            jnp.concatenate([win[:, :, 1:], z], axis=2)]


def _accum_rows(win, w, o_ref, bias, relu, TR, Cout, W, out_dtype):
    Cin = win.shape[1]
    shifts = _shift3(win, W)
    for r in range(TR):
        acc = jnp.zeros((Cout, W), F32)
        for kx in range(3):
            rhs = shifts[kx][r:r + 3].reshape(3 * Cin, W)
            acc = acc + _dot(w[kx], rhs)
        acc = acc + bias
        if relu:
            acc = jnp.maximum(acc, 0.0)
        o_ref[0, r] = acc.astype(out_dtype)


def _conv_s1_body(*refs, nin, TR, W, Cout, relu, out_dtype):
    o_ref = refs[-1]
    b_ref = refs[-2]
    wins = []
    for k in range(nin):
        top, cur, bot = refs[3 * k], refs[3 * k + 1], refs[3 * k + 2]
        wins.append(jnp.concatenate([top[0], cur[0], bot[0]], axis=0))
    win = wins[0] if nin == 1 else jnp.concatenate(wins, axis=1)
    _accum_rows(win, refs[3 * nin], o_ref, b_ref[...], relu, TR, Cout, W,
                out_dtype)


def _halos(x, TR):
    B, H, C, W = x.shape
    z = jnp.zeros((B, 1, C, W), x.dtype)
    top = jnp.concatenate([z, x[:, TR - 1::TR][:, :-1]], axis=1)
    bot = jnp.concatenate([x[:, TR::TR], z], axis=1)
    return top, bot


def _conv_s1(xs, w, b, relu, TR, out_dtype=BF16):
    B, H, _, W = xs[0].shape
    Cout = w.shape[1]
    nb = H // TR
    args, in_specs = [], []
    for x in xs:
        Cin = x.shape[2]
        top, bot = _halos(x, TR)
        args += [top, x, bot]
        in_specs += [
            pl.BlockSpec((1, 1, Cin, W), lambda bb, i: (bb, i, 0, 0)),
            pl.BlockSpec((1, TR, Cin, W), lambda bb, i: (bb, i, 0, 0)),
            pl.BlockSpec((1, 1, Cin, W), lambda bb, i: (bb, i, 0, 0)),
        ]
    args += [w, b]
    in_specs += [pl.BlockSpec(w.shape, lambda bb, i: (0, 0, 0)),
                 pl.BlockSpec(b.shape, lambda bb, i: (0, 0))]
    return pl.pallas_call(
        functools.partial(_conv_s1_body, nin=len(xs), TR=TR, W=W, Cout=Cout,
                          relu=relu, out_dtype=out_dtype),
        grid=(B, nb),
        in_specs=in_specs,
        out_specs=pl.BlockSpec((1, TR, Cout, W), lambda bb, i: (bb, i, 0, 0)),
        out_shape=jax.ShapeDtypeStruct((B, H, Cout, W), out_dtype),
    )(*args)


def _conv_s2_body(xc, xb, s0, s1, w, b_ref, o_ref, *, TR, Wo, Cin, Cout):
    W = 2 * Wo
    z = jnp.zeros((1, Cin, W), BF16)
    win = jnp.concatenate([xc[0], xb[0], z], axis=0)
    flat = win.reshape((2 * TR + 2) * Cin, W)
    wine = _dot(flat, s0[...]).astype(BF16).reshape(2 * TR + 2, Cin, Wo)
    wino = _dot(flat, s1[...]).astype(BF16).reshape(2 * TR + 2, Cin, Wo)
    zc = jnp.zeros((Cout, 1), F32)
    for r in range(TR):
        rhs_e = wine[2 * r:2 * r + 3].reshape(3 * Cin, Wo)
        rhs_o = wino[2 * r:2 * r + 3].reshape(3 * Cin, Wo)
        f0 = _dot(w[0], rhs_e)
        f1 = _dot(w[1], rhs_o)
        f2 = _dot(w[2], rhs_e)
        acc = f0 + f1 + jnp.concatenate([f2[:, 1:], zc], axis=1) + b_ref[...]
        acc = jnp.maximum(acc, 0.0)
        o_ref[0, r] = acc.astype(BF16)


def _conv_s2(x, w, b, TR):
    B, H, Cin, W = x.shape
    Cout = w.shape[1]
    Ho, Wo = H // 2, W // 2
    nb = Ho // TR
    zb = jnp.zeros((B, 1, Cin, W), BF16)
    bot = jnp.concatenate([x[:, 2 * TR::2 * TR], zb], axis=1)
    cols = jnp.arange(W)[:, None]
    s0 = (cols == 2 * jnp.arange(Wo)[None, :]).astype(BF16)
    s1 = (cols == 2 * jnp.arange(Wo)[None, :] + 1).astype(BF16)
    return pl.pallas_call(
        functools.partial(_conv_s2_body, TR=TR, Wo=Wo, Cin=Cin, Cout=Cout),
        grid=(B, nb),
        in_specs=[pl.BlockSpec((1, 2 * TR, Cin, W), lambda bb, i: (bb, i, 0, 0)),
                  pl.BlockSpec((1, 1, Cin, W), lambda bb, i: (bb, i, 0, 0)),
                  pl.BlockSpec(s0.shape, lambda bb, i: (0, 0)),
                  pl.BlockSpec(s1.shape, lambda bb, i: (0, 0)),
                  pl.BlockSpec(w.shape, lambda bb, i: (0, 0, 0)),
                  pl.BlockSpec(b.shape, lambda bb, i: (0, 0))],
        out_specs=pl.BlockSpec((1, TR, Cout, Wo), lambda bb, i: (bb, i, 0, 0)),
        out_shape=jax.ShapeDtypeStruct((B, Ho, Cout, Wo), BF16),
    )(x, bot, s0, s1, w, b)


def _conv_up_body(top, cur, bot, rmat, w, b_ref, o_ref, *, TR2, Wd, Cin, Cout):
    TR, W = 2 * TR2, 2 * Wd
    dwin = jnp.concatenate([top[0], cur[0], bot[0]], axis=0)
    flat = dwin.reshape((TR2 + 2) * Cin, Wd)
    wide = _dot(flat, rmat[...]).astype(BF16).reshape(TR2 + 2, Cin, W)
    urep = jnp.repeat(wide, 2, axis=0)[1:TR + 3]
    _accum_rows(urep, w, o_ref, b_ref[...], True, TR, Cout, W, BF16)


def _conv_up(d, w, b, TR2):
    B, Hd, Cin, Wd = d.shape
    Cout = w.shape[1]
    H, W = 2 * Hd, 2 * Wd
    TR = 2 * TR2
    nb = Hd // TR2
    top, bot = _halos(d, TR2)
    rmat = (jnp.arange(Wd)[:, None] == (jnp.arange(W)[None, :] // 2)).astype(BF16)
    return pl.pallas_call(
        functools.partial(_conv_up_body, TR2=TR2, Wd=Wd, Cin=Cin, Cout=Cout),
        grid=(B, nb),
        in_specs=[pl.BlockSpec((1, 1, Cin, Wd), lambda bb, i: (bb, i, 0, 0)),
                  pl.BlockSpec((1, TR2, Cin, Wd), lambda bb, i: (bb, i, 0, 0)),
                  pl.BlockSpec((1, 1, Cin, Wd), lambda bb, i: (bb, i, 0, 0)),
                  pl.BlockSpec(rmat.shape, lambda bb, i: (0, 0)),
                  pl.BlockSpec(w.shape, lambda bb, i: (0, 0, 0)),
                  pl.BlockSpec(b.shape, lambda bb, i: (0, 0))],
        out_specs=pl.BlockSpec((1, TR, Cout, W), lambda bb, i: (bb, i, 0, 0)),
        out_shape=jax.ShapeDtypeStruct((B, H, Cout, W), BF16),
    )(top, d, bot, rmat, w, b)


def _wk(w):
    O, I = w.shape[0], w.shape[1]
    return jnp.transpose(w, (3, 0, 2, 1)).reshape(3, O, 3 * I).astype(BF16)


def kernel(image, w1, b1, w2, b2, w3, b3, w4, b4, w5, b5, w6, b6, w_out, b_out):
    x = jnp.transpose(image, (0, 2, 1, 3)).astype(BF16)
    bb = lambda b: b.reshape(-1, 1).astype(F32)
    t = _conv_s1([x], _wk(w1), bb(b1), True, 32)
    e1 = _conv_s1([t], _wk(w2), bb(b2), True, 32)
    d = _conv_s2(e1, _wk(w3), bb(b3), 32)
    d = _conv_s1([d], _wk(w4), bb(b4), True, 32)
    u = _conv_up(d, _wk(w5), bb(b5), 16)
    f = _conv_s1([u, e1], _wk(w6), bb(b6), True, 32)
    y = _conv_s1([f], _wk(w_out), bb(b_out), True, 32, out_dtype=F32)
    return jnp.transpose(y, (0, 2, 1, 3))


# speedup vs baseline: 1.0855x; 1.0581x over previous
import functools

import jax
import jax.numpy as jnp
from jax.experimental import pallas as pl

BF16 = jnp.bfloat16
F32 = jnp.float32


def _dot(a, b):
    return jax.lax.dot_general(a, b, (((1,), (0,)), ((), ())),
                               preferred_element_type=F32)


def _shift3(win, W):
    z = jnp.zeros(win.shape[:2] + (1,), win.dtype)
    return [jnp.concatenate([z, win[:, :, :W - 1]], axis=2),
            win,
            jnp.concatenate([win[:, :, 1:], z], axis=2)]


def _accum_rows(win, w, o_ref, bias, relu, TR, Cout, W, out_dtype):
    Cin = win.shape[1]
    shifts = _shift3(win, W)
    for r in range(TR):
        acc = jnp.zeros((Cout, W), F32)
        for kx in range(3):
            rhs = shifts[kx][r:r + 3].reshape(3 * Cin, W)
            acc = acc + _dot(w[kx], rhs)
        acc = acc + bias
        if relu:
            acc = jnp.maximum(acc, 0.0)
        o_ref[0, r] = acc.astype(out_dtype)


def _conv_s1_body(*refs, nin, TR, W, Cout, relu, out_dtype):
    o_ref = refs[-1]
    b_ref = refs[-2]
    wins = []
    for k in range(nin):
        top, cur, bot = refs[3 * k], refs[3 * k + 1], refs[3 * k + 2]
        wins.append(jnp.concatenate([top[0], cur[0], bot[0]], axis=0))
    win = wins[0] if nin == 1 else jnp.concatenate(wins, axis=1)
    _accum_rows(win, refs[3 * nin], o_ref, b_ref[...], relu, TR, Cout, W,
                out_dtype)


def _halos(x, TR):
    B, H, C, W = x.shape
    z = jnp.zeros((B, 1, C, W), x.dtype)
    top = jnp.concatenate([z, x[:, TR - 1::TR][:, :-1]], axis=1)
    bot = jnp.concatenate([x[:, TR::TR], z], axis=1)
    return top, bot


def _conv_s1(xs, w, b, relu, TR, out_dtype=BF16):
    B, H, _, W = xs[0].shape
    Cout = w.shape[1]
    nb = H // TR
    args, in_specs = [], []
    for x in xs:
        Cin = x.shape[2]
        top, bot = _halos(x, TR)
        args += [top, x, bot]
        in_specs += [
            pl.BlockSpec((1, 1, Cin, W), lambda bb, i: (bb, i, 0, 0)),
            pl.BlockSpec((1, TR, Cin, W), lambda bb, i: (bb, i, 0, 0)),
            pl.BlockSpec((1, 1, Cin, W), lambda bb, i: (bb, i, 0, 0)),
        ]
    args += [w, b]
    in_specs += [pl.BlockSpec(w.shape, lambda bb, i: (0, 0, 0)),
                 pl.BlockSpec(b.shape, lambda bb, i: (0, 0))]
    return pl.pallas_call(
        functools.partial(_conv_s1_body, nin=len(xs), TR=TR, W=W, Cout=Cout,
                          relu=relu, out_dtype=out_dtype),
        grid=(B, nb),
        in_specs=in_specs,
        out_specs=pl.BlockSpec((1, TR, Cout, W), lambda bb, i: (bb, i, 0, 0)),
        out_shape=jax.ShapeDtypeStruct((B, H, Cout, W), out_dtype),
    )(*args)


def _conv_s2_body(xc, xb, s0, s1, w, b_ref, o_ref, *, TR, Wo, Cin, Cout):
    W = 2 * Wo
    z = jnp.zeros((1, Cin, W), BF16)
    win = jnp.concatenate([xc[0], xb[0], z], axis=0)
    flat = win.reshape((2 * TR + 2) * Cin, W)
    wine = _dot(flat, s0[...]).astype(BF16).reshape(2 * TR + 2, Cin, Wo)
    wino = _dot(flat, s1[...]).astype(BF16).reshape(2 * TR + 2, Cin, Wo)
    zc = jnp.zeros((Cout, 1), F32)
    for r in range(TR):
        rhs_e = wine[2 * r:2 * r + 3].reshape(3 * Cin, Wo)
        rhs_o = wino[2 * r:2 * r + 3].reshape(3 * Cin, Wo)
        f0 = _dot(w[0], rhs_e)
        f1 = _dot(w[1], rhs_o)
        f2 = _dot(w[2], rhs_e)
        acc = f0 + f1 + jnp.concatenate([f2[:, 1:], zc], axis=1) + b_ref[...]
        acc = jnp.maximum(acc, 0.0)
        o_ref[0, r] = acc.astype(BF16)


def _conv_s2(x, w, b, TR):
    B, H, Cin, W = x.shape
    Cout = w.shape[1]
    Ho, Wo = H // 2, W // 2
    nb = Ho // TR
    zb = jnp.zeros((B, 1, Cin, W), BF16)
    bot = jnp.concatenate([x[:, 2 * TR::2 * TR], zb], axis=1)
    cols = jnp.arange(W)[:, None]
    s0 = (cols == 2 * jnp.arange(Wo)[None, :]).astype(BF16)
    s1 = (cols == 2 * jnp.arange(Wo)[None, :] + 1).astype(BF16)
    return pl.pallas_call(
        functools.partial(_conv_s2_body, TR=TR, Wo=Wo, Cin=Cin, Cout=Cout),
        grid=(B, nb),
        in_specs=[pl.BlockSpec((1, 2 * TR, Cin, W), lambda bb, i: (bb, i, 0, 0)),
                  pl.BlockSpec((1, 1, Cin, W), lambda bb, i: (bb, i, 0, 0)),
                  pl.BlockSpec(s0.shape, lambda bb, i: (0, 0)),
                  pl.BlockSpec(s1.shape, lambda bb, i: (0, 0)),
                  pl.BlockSpec(w.shape, lambda bb, i: (0, 0, 0)),
                  pl.BlockSpec(b.shape, lambda bb, i: (0, 0))],
        out_specs=pl.BlockSpec((1, TR, Cout, Wo), lambda bb, i: (bb, i, 0, 0)),
        out_shape=jax.ShapeDtypeStruct((B, Ho, Cout, Wo), BF16),
    )(x, bot, s0, s1, w, b)


def _conv_up_body(top, cur, bot, rmat, w, b_ref, o_ref, *, TR2, Wd, Cin, Cout):
    TR, W = 2 * TR2, 2 * Wd
    dwin = jnp.concatenate([top[0], cur[0], bot[0]], axis=0)
    flat = dwin.reshape((TR2 + 2) * Cin, Wd)
    wide = _dot(flat, rmat[...]).astype(BF16).reshape(TR2 + 2, Cin, W)
    urep = jnp.repeat(wide, 2, axis=0)[1:TR + 3]
    _accum_rows(urep, w, o_ref, b_ref[...], True, TR, Cout, W, BF16)


def _conv_up(d, w, b, TR2):
    B, Hd, Cin, Wd = d.shape
    Cout = w.shape[1]
    H, W = 2 * Hd, 2 * Wd
    TR = 2 * TR2
    nb = Hd // TR2
    top, bot = _halos(d, TR2)
    rmat = (jnp.arange(Wd)[:, None] == (jnp.arange(W)[None, :] // 2)).astype(BF16)
    return pl.pallas_call(
        functools.partial(_conv_up_body, TR2=TR2, Wd=Wd, Cin=Cin, Cout=Cout),
        grid=(B, nb),
        in_specs=[pl.BlockSpec((1, 1, Cin, Wd), lambda bb, i: (bb, i, 0, 0)),
                  pl.BlockSpec((1, TR2, Cin, Wd), lambda bb, i: (bb, i, 0, 0)),
                  pl.BlockSpec((1, 1, Cin, Wd), lambda bb, i: (bb, i, 0, 0)),
                  pl.BlockSpec(rmat.shape, lambda bb, i: (0, 0)),
                  pl.BlockSpec(w.shape, lambda bb, i: (0, 0, 0)),
                  pl.BlockSpec(b.shape, lambda bb, i: (0, 0))],
        out_specs=pl.BlockSpec((1, TR, Cout, W), lambda bb, i: (bb, i, 0, 0)),
        out_shape=jax.ShapeDtypeStruct((B, H, Cout, W), BF16),
    )(top, d, bot, rmat, w, b)


def _wk(w):
    O, I = w.shape[0], w.shape[1]
    return jnp.transpose(w, (3, 0, 2, 1)).reshape(3, O, 3 * I).astype(BF16)


def kernel(image, w1, b1, w2, b2, w3, b3, w4, b4, w5, b5, w6, b6, w_out, b_out):
    x = jnp.transpose(image, (0, 2, 1, 3)).astype(BF16)
    bb = lambda b: b.reshape(-1, 1).astype(F32)
    t = _conv_s1([x], _wk(w1), bb(b1), True, 64)
    e1 = _conv_s1([t], _wk(w2), bb(b2), True, 64)
    d = _conv_s2(e1, _wk(w3), bb(b3), 64)
    d = _conv_s1([d], _wk(w4), bb(b4), True, 64)
    u = _conv_up(d, _wk(w5), bb(b5), 32)
    f = _conv_s1([u, e1], _wk(w6), bb(b6), True, 64)
    y = _conv_s1([f], _wk(w_out), bb(b_out), True, 64, out_dtype=F32)
    return jnp.transpose(y, (0, 2, 1, 3))


# speedup vs baseline: 1.1019x; 1.0151x over previous
import functools

import jax
import jax.numpy as jnp
from jax.experimental import pallas as pl
from jax.experimental.pallas import tpu as pltpu

BF16 = jnp.bfloat16
F32 = jnp.float32


def _dot(a, b):
    return jax.lax.dot_general(a, b, (((1,), (0,)), ((), ())),
                               preferred_element_type=F32)


def _shift3(win, W):
    z = jnp.zeros(win.shape[:2] + (1,), win.dtype)
    return [jnp.concatenate([z, win[:, :, :W - 1]], axis=2),
            win,
            jnp.concatenate([win[:, :, 1:], z], axis=2)]


def _accum_rows(win, w, o_ref, bias, relu, TR, Cout, W, out_dtype):
    Cin = win.shape[1]
    shifts = _shift3(win, W)
    for r in range(TR):
        acc = jnp.zeros((Cout, W), F32)
        for kx in range(3):
            rhs = shifts[kx][r:r + 3].reshape(3 * Cin, W)
            acc = acc + _dot(w[kx], rhs)
        acc = acc + bias
        if relu:
            acc = jnp.maximum(acc, 0.0)
        o_ref[0, r] = acc.astype(out_dtype)


def _conv_s1_body(*refs, nin, TR, W, Cout, relu, out_dtype):
    o_ref = refs[-1]
    b_ref = refs[-2]
    wins = []
    for k in range(nin):
        top, cur, bot = refs[3 * k], refs[3 * k + 1], refs[3 * k + 2]
        wins.append(jnp.concatenate([top[0], cur[0], bot[0]], axis=0))
    win = wins[0] if nin == 1 else jnp.concatenate(wins, axis=1)
    _accum_rows(win, refs[3 * nin], o_ref, b_ref[...], relu, TR, Cout, W,
                out_dtype)


def _halos(x, TR):
    B, H, C, W = x.shape
    z = jnp.zeros((B, 1, C, W), x.dtype)
    top = jnp.concatenate([z, x[:, TR - 1::TR][:, :-1]], axis=1)
    bot = jnp.concatenate([x[:, TR::TR], z], axis=1)
    return top, bot


def _conv_s1(xs, w, b, relu, TR, out_dtype=BF16):
    B, H, _, W = xs[0].shape
    Cout = w.shape[1]
    nb = H // TR
    args, in_specs = [], []
    for x in xs:
        Cin = x.shape[2]
        top, bot = _halos(x, TR)
        args += [top, x, bot]
        in_specs += [
            pl.BlockSpec((1, 1, Cin, W), lambda bb, i: (bb, i, 0, 0)),
            pl.BlockSpec((1, TR, Cin, W), lambda bb, i: (bb, i, 0, 0)),
            pl.BlockSpec((1, 1, Cin, W), lambda bb, i: (bb, i, 0, 0)),
        ]
    args += [w, b]
    in_specs += [pl.BlockSpec(w.shape, lambda bb, i: (0, 0, 0)),
                 pl.BlockSpec(b.shape, lambda bb, i: (0, 0))]
    return pl.pallas_call(
        functools.partial(_conv_s1_body, nin=len(xs), TR=TR, W=W, Cout=Cout,
                          relu=relu, out_dtype=out_dtype),
        grid=(B, nb),
        in_specs=in_specs,
        out_specs=pl.BlockSpec((1, TR, Cout, W), lambda bb, i: (bb, i, 0, 0)),
        out_shape=jax.ShapeDtypeStruct((B, H, Cout, W), out_dtype),
    )(*args)


def _conv_s2_body(xc, xb, s0, s1, w, b_ref, o_ref, *, TR, Wo, Cin, Cout):
    W = 2 * Wo
    z = jnp.zeros((1, Cin, W), BF16)
    win = jnp.concatenate([xc[0], xb[0], z], axis=0)
    flat = win.reshape((2 * TR + 2) * Cin, W)
    wine = _dot(flat, s0[...]).astype(BF16).reshape(2 * TR + 2, Cin, Wo)
    wino = _dot(flat, s1[...]).astype(BF16).reshape(2 * TR + 2, Cin, Wo)
    zc = jnp.zeros((Cout, 1), F32)
    for r in range(TR):
        rhs_e = wine[2 * r:2 * r + 3].reshape(3 * Cin, Wo)
        rhs_o = wino[2 * r:2 * r + 3].reshape(3 * Cin, Wo)
        f0 = _dot(w[0], rhs_e)
        f1 = _dot(w[1], rhs_o)
        f2 = _dot(w[2], rhs_e)
        acc = f0 + f1 + jnp.concatenate([f2[:, 1:], zc], axis=1) + b_ref[...]
        acc = jnp.maximum(acc, 0.0)
        o_ref[0, r] = acc.astype(BF16)


def _conv_s2(x, w, b, TR):
    B, H, Cin, W = x.shape
    Cout = w.shape[1]
    Ho, Wo = H // 2, W // 2
    nb = Ho // TR
    zb = jnp.zeros((B, 1, Cin, W), BF16)
    bot = jnp.concatenate([x[:, 2 * TR::2 * TR], zb], axis=1)
    cols = jnp.arange(W)[:, None]
    s0 = (cols == 2 * jnp.arange(Wo)[None, :]).astype(BF16)
    s1 = (cols == 2 * jnp.arange(Wo)[None, :] + 1).astype(BF16)
    return pl.pallas_call(
        functools.partial(_conv_s2_body, TR=TR, Wo=Wo, Cin=Cin, Cout=Cout),
        grid=(B, nb),
        in_specs=[pl.BlockSpec((1, 2 * TR, Cin, W), lambda bb, i: (bb, i, 0, 0)),
                  pl.BlockSpec((1, 1, Cin, W), lambda bb, i: (bb, i, 0, 0)),
                  pl.BlockSpec(s0.shape, lambda bb, i: (0, 0)),
                  pl.BlockSpec(s1.shape, lambda bb, i: (0, 0)),
                  pl.BlockSpec(w.shape, lambda bb, i: (0, 0, 0)),
                  pl.BlockSpec(b.shape, lambda bb, i: (0, 0))],
        out_specs=pl.BlockSpec((1, TR, Cout, Wo), lambda bb, i: (bb, i, 0, 0)),
        out_shape=jax.ShapeDtypeStruct((B, Ho, Cout, Wo), BF16),
    )(x, bot, s0, s1, w, b)


def _conv_up_body(top, cur, bot, rmat, w, b_ref, o_ref, *, TR2, Wd, Cin, Cout):
    TR, W = 2 * TR2, 2 * Wd
    dwin = jnp.concatenate([top[0], cur[0], bot[0]], axis=0)
    flat = dwin.reshape((TR2 + 2) * Cin, Wd)
    wide = _dot(flat, rmat[...]).astype(BF16).reshape(TR2 + 2, Cin, W)
    urep = jnp.repeat(wide, 2, axis=0)[1:TR + 3]
    _accum_rows(urep, w, o_ref, b_ref[...], True, TR, Cout, W, BF16)


def _conv_up(d, w, b, TR2):
    B, Hd, Cin, Wd = d.shape
    Cout = w.shape[1]
    H, W = 2 * Hd, 2 * Wd
    TR = 2 * TR2
    nb = Hd // TR2
    top, bot = _halos(d, TR2)
    rmat = (jnp.arange(Wd)[:, None] == (jnp.arange(W)[None, :] // 2)).astype(BF16)
    return pl.pallas_call(
        functools.partial(_conv_up_body, TR2=TR2, Wd=Wd, Cin=Cin, Cout=Cout),
        grid=(B, nb),
        in_specs=[pl.BlockSpec((1, 1, Cin, Wd), lambda bb, i: (bb, i, 0, 0)),
                  pl.BlockSpec((1, TR2, Cin, Wd), lambda bb, i: (bb, i, 0, 0)),
                  pl.BlockSpec((1, 1, Cin, Wd), lambda bb, i: (bb, i, 0, 0)),
                  pl.BlockSpec(rmat.shape, lambda bb, i: (0, 0)),
                  pl.BlockSpec(w.shape, lambda bb, i: (0, 0, 0)),
                  pl.BlockSpec(b.shape, lambda bb, i: (0, 0))],
        out_specs=pl.BlockSpec((1, TR, Cout, W), lambda bb, i: (bb, i, 0, 0)),
        out_shape=jax.ShapeDtypeStruct((B, H, Cout, W), BF16),
    )(top, d, bot, rmat, w, b)


def _halos2(x, TR):
    B, H, C, W = x.shape
    nb = H // TR
    z = jnp.zeros((B, 1, C, W), x.dtype)
    ta = jnp.concatenate([z, x[:, TR - 2::TR][:, :-1]], axis=1)
    tb = jnp.concatenate([z, x[:, TR - 1::TR][:, :-1]], axis=1)
    top = jnp.stack([ta, tb], axis=2).reshape(B, 2 * nb, C, W)
    ba = jnp.concatenate([x[:, TR::TR], z], axis=1)
    bb_ = jnp.concatenate([x[:, TR + 1::TR], z], axis=1)
    bot = jnp.stack([ba, bb_], axis=2).reshape(B, 2 * nb, C, W)
    return top, bot


def _conv67_body(ut, uc, ub, et, ec, eb, w6, w7, b6, b7, o_ref, fscr, *,
                 TR, W, Cout):
    win = jnp.concatenate([
        jnp.concatenate([ut[0], uc[0], ub[0]], axis=0),
        jnp.concatenate([et[0], ec[0], eb[0]], axis=0)], axis=1)
    _accum_rows(win, w6, fscr, b6[...], True, TR + 2, Cout, W, BF16)
    i = pl.program_id(1)
    n = pl.num_programs(1)
    tm = jnp.where(i > 0, 1.0, 0.0).astype(BF16)
    bm = jnp.where(i < n - 1, 1.0, 0.0).astype(BF16)
    fwin = jnp.concatenate([fscr[0, 0:1] * tm, fscr[0, 1:TR + 1],
                            fscr[0, TR + 1:TR + 2] * bm], axis=0)
    shifts = _shift3(fwin, W)
    zc = jnp.zeros((1, 1), F32)
    for r in range(TR):
        acc = jnp.zeros((1, W), F32)
        for kx in range(3):
            rhs = shifts[kx][r:r + 3].reshape(3 * Cout, W)
            acc = acc + _dot(w7[kx], rhs)
        acc = jnp.maximum(acc + b7[...], 0.0)
        o_ref[0, r] = acc


def _conv67(u, e1, w6, b6, w7, b7, TR):
    B, H, Cin, W = u.shape
    Cout = w6.shape[1]
    nb = H // TR
    ut, ub = _halos2(u, TR)
    et, eb = _halos2(e1, TR)
    two = lambda: pl.BlockSpec((1, 2, Cin, W), lambda bb, i: (bb, i, 0, 0))
    big = lambda: pl.BlockSpec((1, TR, Cin, W), lambda bb, i: (bb, i, 0, 0))
    return pl.pallas_call(
        functools.partial(_conv67_body, TR=TR, W=W, Cout=Cout),
        grid=(B, nb),
        in_specs=[two(), big(), two(), two(), big(), two(),
                  pl.BlockSpec(w6.shape, lambda bb, i: (0, 0, 0)),
                  pl.BlockSpec(w7.shape, lambda bb, i: (0, 0, 0)),
                  pl.BlockSpec(b6.shape, lambda bb, i: (0, 0)),
                  pl.BlockSpec(b7.shape, lambda bb, i: (0, 0))],
        out_specs=pl.BlockSpec((1, TR, 1, W), lambda bb, i: (bb, i, 0, 0)),
        out_shape=jax.ShapeDtypeStruct((B, H, 1, W), F32),
        scratch_shapes=[pltpu.VMEM((1, TR + 2, Cout, W), BF16)],
    )(ut, u, ub, et, e1, eb, w6, w7, b6, b7)


def _wk(w):
    O, I = w.shape[0], w.shape[1]
    return jnp.transpose(w, (3, 0, 2, 1)).reshape(3, O, 3 * I).astype(BF16)


def kernel(image, w1, b1, w2, b2, w3, b3, w4, b4, w5, b5, w6, b6, w_out, b_out):
    x = jnp.transpose(image, (0, 2, 1, 3)).astype(BF16)
    bb = lambda b: b.reshape(-1, 1).astype(F32)
    t = _conv_s1([x], _wk(w1), bb(b1), True, 128)
    e1 = _conv_s1([t], _wk(w2), bb(b2), True, 64)
    d = _conv_s2(e1, _wk(w3), bb(b3), 64)
    d = _conv_s1([d], _wk(w4), bb(b4), True, 64)
    u = _conv_up(d, _wk(w5), bb(b5), 32)
    y = _conv67(u, e1, _wk(w6), bb(b6), _wk(w_out), bb(b_out), 64)
    return jnp.transpose(y, (0, 2, 1, 3))
